# baseline (speedup 1.0000x reference)
import functools

import jax
import jax.numpy as jnp
from jax import lax
from jax.experimental import pallas as pl
from jax.experimental.pallas import tpu as pltpu
from jax.experimental.pallas import tpu_sc as plsc

_D = 32
_CODES = 8192
_TOKENS = 8192
_TILE_T = 256
_NT = _TOKENS // _TILE_T

_NC = 2
_NS = 16
_NW = _NC * _NS
_GW = 128
_CHUNKS = _TOKENS // (_NW * _GW)


_NLV = 16


@functools.cache
def _make_sc_gather():
    @functools.partial(
        pl.kernel,
        out_type=[
            jax.ShapeDtypeStruct((_TOKENS, _D), jnp.float32),
            jax.ShapeDtypeStruct((_NW, _NLV), jnp.float32),
        ],
        mesh=plsc.VectorSubcoreMesh(core_axis_name="c", subcore_axis_name="s"),
        scratch_types=[
            pltpu.VMEM((_GW,), jnp.int32),
            pltpu.VMEM((_GW, _D), jnp.float32),
            pltpu.VMEM((_GW, _D), jnp.float32),
            pltpu.VMEM((_NLV,), jnp.float32),
        ],
        compiler_params=pltpu.CompilerParams(use_tc_tiling_on_sc=False),
    )
    def _sc_gather(table_hbm, idx_hbm, z_hbm, out_hbm, part_hbm,
                   idx_v, rows_v, z_v, acc_v):
        wid = lax.axis_index("s") * _NC + lax.axis_index("c")
        acc_v[...] = jnp.zeros((_NLV,), jnp.float32)
        for j in range(_CHUNKS):
            r = wid * _CHUNKS + j
            pltpu.sync_copy(idx_hbm.at[r], idx_v)
            pltpu.sync_copy(table_hbm.at[idx_v], rows_v)
            pltpu.sync_copy(rows_v, out_hbm.at[pl.ds(r * _GW, _GW)])
            pltpu.sync_copy(z_hbm.at[pl.ds(r * _GW, _GW)], z_v)

            @pl.loop(0, _GW)
            def _(t):
                for c in range(_D // _NLV):
                    d = (rows_v[t, pl.ds(c * _NLV, _NLV)]
                         - z_v[t, pl.ds(c * _NLV, _NLV)])
                    acc_v[...] = acc_v[...] + d * d

        pltpu.sync_copy(acc_v, part_hbm.at[wid])

    return _sc_gather


def _loss_body(q_ref, z_ref, out_ref):
    d = q_ref[...] - z_ref[...]
    out_ref[0, :, 0] = jnp.sum(d * d, axis=1)


_loss_partials = pl.pallas_call(
    _loss_body,
    grid=(_NT,),
    in_specs=[
        pl.BlockSpec((_TILE_T, _D), lambda i: (i, 0)),
        pl.BlockSpec((_TILE_T, _D), lambda i: (i, 0)),
    ],
    out_specs=pl.BlockSpec((1, _TILE_T, 1), lambda i: (i, 0, 0)),
    out_shape=jax.ShapeDtypeStruct((_NT, _TILE_T, 1), jnp.float32),
)


def kernel(z, embeddings):
    input_shape = z.shape
    flat_z = z.reshape(-1, _D)
    z_norm = jnp.sum(flat_z ** 2, axis=1, keepdims=True)
    e_norm = jnp.sum(embeddings ** 2, axis=1)
    dist = z_norm + e_norm - 2.0 * jnp.matmul(flat_z, embeddings.T)
    idx_flat = jnp.argmin(dist, axis=1).astype(jnp.int32)

    quant_flat, partials = _make_sc_gather()(
        embeddings, idx_flat.reshape(_NW * _CHUNKS, _GW), flat_z)
    quantized = quant_flat.reshape(input_shape)
    vq_loss = 1.5 * jnp.sum(partials) / (_TOKENS * _D)

    quantized_st = z + (quantized - z)
    idx_out = idx_flat.reshape((input_shape[0],) + tuple(input_shape[2:]))
    return (quantized_st, vq_loss, idx_out)


# speedup vs baseline: 1.0150x; 1.0150x over previous
import functools

import jax
import jax.numpy as jnp
from jax import lax
from jax.experimental import pallas as pl
from jax.experimental.pallas import tpu as pltpu
from jax.experimental.pallas import tpu_sc as plsc

_D = 32
_CODES = 8192
_TOKENS = 8192
_TILE_T = 256
_NT = _TOKENS // _TILE_T

_NC = 2
_NS = 16
_NW = _NC * _NS
_GW = 128
_CHUNKS = _TOKENS // (_NW * _GW)


_NLV = 16


@functools.cache
def _make_sc_gather():
    @functools.partial(
        pl.kernel,
        out_type=[
            jax.ShapeDtypeStruct((_TOKENS, _D), jnp.float32),
            jax.ShapeDtypeStruct((_NW, _NLV), jnp.float32),
        ],
        mesh=plsc.VectorSubcoreMesh(core_axis_name="c", subcore_axis_name="s"),
        scratch_types=[
            pltpu.VMEM((_CHUNKS, _GW), jnp.int32),
            pltpu.VMEM((_CHUNKS, _GW, _D), jnp.float32),
            pltpu.VMEM((_CHUNKS, _GW, _D), jnp.float32),
            pltpu.VMEM((_NLV,), jnp.float32),
            pltpu.SemaphoreType.DMA,
            pltpu.SemaphoreType.DMA,
        ],
        compiler_params=pltpu.CompilerParams(use_tc_tiling_on_sc=False),
    )
    def _sc_gather(table_hbm, idx_hbm, z_hbm, out_hbm, part_hbm,
                   idx_v, rows_v, z_v, acc_v, gsem, zsem):
        wid = lax.axis_index("s") * _NC + lax.axis_index("c")
        pltpu.sync_copy(idx_hbm.at[wid], idx_v)
        gathers = []
        zloads = []
        for j in range(_CHUNKS):
            r = wid * _CHUNKS + j
            gathers.append(pltpu.async_copy(
                table_hbm.at[idx_v.at[j]], rows_v.at[j], gsem))
            zloads.append(pltpu.async_copy(
                z_hbm.at[pl.ds(r * _GW, _GW)], z_v.at[j], zsem))
        acc_v[...] = jnp.zeros((_NLV,), jnp.float32)
        for j in range(_CHUNKS):
            r = wid * _CHUNKS + j
            gathers[j].wait()
            pltpu.sync_copy(rows_v.at[j], out_hbm.at[pl.ds(r * _GW, _GW)])
            zloads[j].wait()

            @pl.loop(0, _GW)
            def _(t):
                for c in range(_D // _NLV):
                    d = (rows_v[j, t, pl.ds(c * _NLV, _NLV)]
                         - z_v[j, t, pl.ds(c * _NLV, _NLV)])
                    acc_v[...] = acc_v[...] + d * d

        pltpu.sync_copy(acc_v, part_hbm.at[wid])

    return _sc_gather


def _loss_body(q_ref, z_ref, out_ref):
    d = q_ref[...] - z_ref[...]
    out_ref[0, :, 0] = jnp.sum(d * d, axis=1)


_loss_partials = pl.pallas_call(
    _loss_body,
    grid=(_NT,),
    in_specs=[
        pl.BlockSpec((_TILE_T, _D), lambda i: (i, 0)),
        pl.BlockSpec((_TILE_T, _D), lambda i: (i, 0)),
    ],
    out_specs=pl.BlockSpec((1, _TILE_T, 1), lambda i: (i, 0, 0)),
    out_shape=jax.ShapeDtypeStruct((_NT, _TILE_T, 1), jnp.float32),
)


def kernel(z, embeddings):
    input_shape = z.shape
    flat_z = z.reshape(-1, _D)
    z_norm = jnp.sum(flat_z ** 2, axis=1, keepdims=True)
    e_norm = jnp.sum(embeddings ** 2, axis=1)
    dist = z_norm + e_norm - 2.0 * jnp.matmul(flat_z, embeddings.T)
    idx_flat = jnp.argmin(dist, axis=1).astype(jnp.int32)

    quant_flat, partials = _make_sc_gather()(
        embeddings, idx_flat.reshape(_NW, _CHUNKS, _GW), flat_z)
    quantized = quant_flat.reshape(input_shape)
    vq_loss = 1.5 * jnp.sum(partials) / (_TOKENS * _D)

    quantized_st = z + (quantized - z)
    idx_out = idx_flat.reshape((input_shape[0],) + tuple(input_shape[2:]))
    return (quantized_st, vq_loss, idx_out)


# speedup vs baseline: 1.0470x; 1.0316x over previous
import functools

import jax
import jax.numpy as jnp
from jax import lax
from jax.experimental import pallas as pl
from jax.experimental.pallas import tpu as pltpu
from jax.experimental.pallas import tpu_sc as plsc

_D = 32
_CODES = 8192
_TOKENS = 8192
_TILE_T = 256
_NT = _TOKENS // _TILE_T

_NC = 2
_NS = 16
_NW = _NC * _NS
_GW = 128
_CHUNKS = _TOKENS // (_NW * _GW)


_NLV = 16


@functools.cache
def _make_sc_gather():
    @functools.partial(
        pl.kernel,
        out_type=[
            jax.ShapeDtypeStruct((_TOKENS, _D), jnp.float32),
            jax.ShapeDtypeStruct((_NW, _NLV), jnp.float32),
        ],
        mesh=plsc.VectorSubcoreMesh(core_axis_name="c", subcore_axis_name="s"),
        scratch_types=[
            pltpu.VMEM((_CHUNKS, _GW), jnp.int32),
            pltpu.VMEM((_CHUNKS, _GW, _D), jnp.float32),
            pltpu.VMEM((_CHUNKS, _GW, _D), jnp.float32),
            pltpu.VMEM((_NLV,), jnp.float32),
            pltpu.SemaphoreType.DMA,
            pltpu.SemaphoreType.DMA,
        ],
        compiler_params=pltpu.CompilerParams(use_tc_tiling_on_sc=False),
    )
    def _sc_gather(table_hbm, idx_hbm, z_hbm, out_hbm, part_hbm,
                   idx_v, rows_v, z_v, acc_v, gsem, zsem):
        wid = lax.axis_index("s") * _NC + lax.axis_index("c")
        pltpu.sync_copy(idx_hbm.at[wid], idx_v)
        gathers = []
        zloads = []
        for j in range(_CHUNKS):
            r = wid * _CHUNKS + j
            gathers.append(pltpu.async_copy(
                table_hbm.at[idx_v.at[j]], rows_v.at[j], gsem))
            zloads.append(pltpu.async_copy(
                z_hbm.at[pl.ds(r * _GW, _GW)], z_v.at[j], zsem))
        acc_v[...] = jnp.zeros((_NLV,), jnp.float32)
        for j in range(_CHUNKS):
            r = wid * _CHUNKS + j
            gathers[j].wait()
            zloads[j].wait()

            @pl.loop(0, _GW)
            def _(t):
                for c in range(_D // _NLV):
                    sl = pl.ds(c * _NLV, _NLV)
                    zv = z_v[j, t, sl]
                    d = rows_v[j, t, sl] - zv
                    acc_v[...] = acc_v[...] + d * d
                    rows_v[j, t, sl] = zv + d

            pltpu.sync_copy(rows_v.at[j], out_hbm.at[pl.ds(r * _GW, _GW)])

        pltpu.sync_copy(acc_v, part_hbm.at[wid])

    return _sc_gather


def _loss_body(q_ref, z_ref, out_ref):
    d = q_ref[...] - z_ref[...]
    out_ref[0, :, 0] = jnp.sum(d * d, axis=1)


_loss_partials = pl.pallas_call(
    _loss_body,
    grid=(_NT,),
    in_specs=[
        pl.BlockSpec((_TILE_T, _D), lambda i: (i, 0)),
        pl.BlockSpec((_TILE_T, _D), lambda i: (i, 0)),
    ],
    out_specs=pl.BlockSpec((1, _TILE_T, 1), lambda i: (i, 0, 0)),
    out_shape=jax.ShapeDtypeStruct((_NT, _TILE_T, 1), jnp.float32),
)


def kernel(z, embeddings):
    input_shape = z.shape
    flat_z = z.reshape(-1, _D)
    z_norm = jnp.sum(flat_z ** 2, axis=1, keepdims=True)
    e_norm = jnp.sum(embeddings ** 2, axis=1)
    dist = z_norm + e_norm - 2.0 * jnp.matmul(flat_z, embeddings.T)
    idx_flat = jnp.argmin(dist, axis=1).astype(jnp.int32)

    qst_flat, partials = _make_sc_gather()(
        embeddings, idx_flat.reshape(_NW, _CHUNKS, _GW), flat_z)
    vq_loss = 1.5 * jnp.sum(partials) / (_TOKENS * _D)

    quantized_st = qst_flat.reshape(input_shape)
    idx_out = idx_flat.reshape((input_shape[0],) + tuple(input_shape[2:]))
    return (quantized_st, vq_loss, idx_out)


# speedup vs baseline: 1.0482x; 1.0011x over previous
import functools

import jax
import jax.numpy as jnp
from jax import lax
from jax.experimental import pallas as pl
from jax.experimental.pallas import tpu as pltpu
from jax.experimental.pallas import tpu_sc as plsc

_D = 32
_CODES = 8192
_TOKENS = 8192

_NC = 2
_NS = 16
_NW = _NC * _NS
_GW = 128
_CHUNKS = _TOKENS // (_NW * _GW)


_NLV = 16


@functools.cache
def _make_sc_gather():
    @functools.partial(
        pl.kernel,
        out_type=[
            jax.ShapeDtypeStruct((_TOKENS, _D), jnp.float32),
            jax.ShapeDtypeStruct((_NW, _NLV), jnp.float32),
        ],
        mesh=plsc.VectorSubcoreMesh(core_axis_name="c", subcore_axis_name="s"),
        scratch_types=[
            pltpu.VMEM((_CHUNKS, _GW), jnp.int32),
            pltpu.VMEM((_CHUNKS, _GW, _D), jnp.float32),
            pltpu.VMEM((_CHUNKS, _GW, _D), jnp.float32),
            pltpu.VMEM((_NLV,), jnp.float32),
            pltpu.SemaphoreType.DMA,
            pltpu.SemaphoreType.DMA,
        ],
        compiler_params=pltpu.CompilerParams(use_tc_tiling_on_sc=False),
    )
    def _sc_gather(table_hbm, idx_hbm, z_hbm, out_hbm, part_hbm,
                   idx_v, rows_v, z_v, acc_v, gsem, zsem):
        wid = lax.axis_index("s") * _NC + lax.axis_index("c")
        pltpu.sync_copy(idx_hbm.at[wid], idx_v)
        gathers = []
        zloads = []
        for j in range(_CHUNKS):
            r = wid * _CHUNKS + j
            gathers.append(pltpu.async_copy(
                table_hbm.at[idx_v.at[j]], rows_v.at[j], gsem))
            zloads.append(pltpu.async_copy(
                z_hbm.at[pl.ds(r * _GW, _GW)], z_v.at[j], zsem))
        acc_v[...] = jnp.zeros((_NLV,), jnp.float32)
        for j in range(_CHUNKS):
            r = wid * _CHUNKS + j
            gathers[j].wait()
            zloads[j].wait()

            @pl.loop(0, _GW)
            def _(t):
                for c in range(_D // _NLV):
                    sl = pl.ds(c * _NLV, _NLV)
                    zv = z_v[j, t, sl]
                    d = rows_v[j, t, sl] - zv
                    acc_v[...] = acc_v[...] + d * d
                    rows_v[j, t, sl] = zv + d

            pltpu.sync_copy(rows_v.at[j], out_hbm.at[pl.ds(r * _GW, _GW)])

        pltpu.sync_copy(acc_v, part_hbm.at[wid])

    return _sc_gather


def kernel(z, embeddings):
    input_shape = z.shape
    flat_z = z.reshape(-1, _D)
    z_norm = jnp.sum(flat_z ** 2, axis=1, keepdims=True)
    e_norm = jnp.sum(embeddings ** 2, axis=1)
    dist = z_norm + e_norm - 2.0 * jnp.matmul(flat_z, embeddings.T)
    idx_flat = jnp.argmin(dist, axis=1).astype(jnp.int32)

    qst_flat, partials = _make_sc_gather()(
        embeddings, idx_flat.reshape(_NW, _CHUNKS, _GW), flat_z)
    vq_loss = 1.5 * jnp.sum(partials) / (_TOKENS * _D)

    quantized_st = qst_flat.reshape(input_shape)
    idx_out = idx_flat.reshape((input_shape[0],) + tuple(input_shape[2:]))
    return (quantized_st, vq_loss, idx_out)
